# baseline (speedup 1.0000x reference)
import functools
import math

import jax
import jax.numpy as jnp
import numpy as np
from jax import lax
from jax.experimental import pallas as pl
from jax.experimental.pallas import tpu as pltpu

HIDDEN = 4096
N_HEADS = 32
N_KV = 8
HD = 128
SEQ = 2048
DQ = N_HEADS * HD
DKV = N_KV * HD
ROPE_THETA = 500000.0
SCALE = 1.0 / math.sqrt(HD)

_INV_FREQ = (1.0 / (ROPE_THETA ** (np.arange(0, HD, 2, dtype=np.float32) / HD))
             ).reshape(1, HD // 2)


def _hadamard_matrix(n):
    H = np.array([[1.0]], dtype=np.float32)
    while H.shape[0] < n:
        H = np.block([[H, H], [H, -H]]).astype(np.float32)
    return H

_HM = _hadamard_matrix(HD)

_DN_T = (((1,), (1,)), ((), ()))
_DN_N = (((1,), (0,)), ((), ()))


def _rope_halves(x1, x2, cos, sin):
    return x1 * cos - x2 * sin, x2 * cos + x1 * sin


def _qkv_kernel(x_ref, wq_ref, wk_ref, wv_ref, hm_ref, if_ref,
                q_ref, k_ref, v_ref, h_ref, acc_ref, *, nk, bs):
    si = pl.program_id(0)
    ki = pl.program_id(1)

    @pl.when(ki == 0)
    def _():
        acc_ref[...] = jnp.zeros_like(acc_ref)

    x = x_ref[...]
    acc_ref[:, :DQ] += lax.dot_general(
        x, wq_ref[...], _DN_T, preferred_element_type=jnp.float32)
    acc_ref[:, DQ:DQ + DKV] += lax.dot_general(
        x, wk_ref[...], _DN_T, preferred_element_type=jnp.float32)
    acc_ref[:, DQ + DKV:] += lax.dot_general(
        x, wv_ref[...], _DN_T, preferred_element_type=jnp.float32)

    @pl.when(ki == nk - 1)
    def _():
        acc = acc_ref[...]
        pos = (si * bs + lax.broadcasted_iota(jnp.int32, (bs, 1), 0)
               ).astype(jnp.float32)
        freqs = pos * if_ref[...]
        cos = jnp.cos(freqs)
        sin = jnp.sin(freqs)
        half = HD // 2
        for h in range(N_HEADS):
            b = h * HD
            r1, r2 = _rope_halves(acc[:, b:b + half], acc[:, b + half:b + HD],
                                  cos, sin)
            q_ref[:, b:b + half] = r1
            q_ref[:, b + half:b + HD] = r2
        hm = hm_ref[...]
        for h in range(N_KV):
            b = DQ + h * HD
            k1, k2 = _rope_halves(acc[:, b:b + half], acc[:, b + half:b + HD],
                                  cos, sin)
            o = h * HD
            k_ref[:, o:o + half] = k1
            k_ref[:, o + half:o + HD] = k2
            kh = jnp.concatenate([k1, k2], axis=1)
            h_ref[:, o:o + HD] = lax.dot_general(
                kh, hm, _DN_N, preferred_element_type=jnp.float32)
        v_ref[...] = acc[:, DQ + DKV:]


def _attn_kernel(q_ref, k_ref, v_ref, o_ref, *, bq, bk):
    qi = pl.program_id(1)
    q = q_ref[...] * SCALE
    nkb = (qi + 1) * (bq // bk)
    neg = jnp.finfo(jnp.float32).min
    row = qi * bq + lax.broadcasted_iota(jnp.int32, (bq, bk), 0)

    def body(j, carry):
        m, l, acc = carry
        kb = k_ref[pl.ds(j * bk, bk), :]
        vb = v_ref[pl.ds(j * bk, bk), :]
        s = lax.dot_general(q, kb, _DN_T, preferred_element_type=jnp.float32)
        col = j * bk + lax.broadcasted_iota(jnp.int32, (bq, bk), 1)
        s = jnp.where(col <= row, s, neg)
        m2 = jnp.maximum(m, jnp.max(s, axis=1, keepdims=True))
        alpha = jnp.exp(m - m2)
        p = jnp.exp(s - m2)
        l2 = l * alpha + jnp.sum(p, axis=1, keepdims=True)
        acc2 = acc * alpha + lax.dot_general(
            p, vb, _DN_N, preferred_element_type=jnp.float32)
        return m2, l2, acc2

    m0 = jnp.full((bq, 1), neg, jnp.float32)
    l0 = jnp.zeros((bq, 1), jnp.float32)
    a0 = jnp.zeros((bq, HD), jnp.float32)
    m, l, acc = lax.fori_loop(0, nkb, body, (m0, l0, a0))
    o_ref[...] = acc / l


def _oproj_kernel(a_ref, w_ref, o_ref, acc_ref, *, nk):
    ki = pl.program_id(1)

    @pl.when(ki == 0)
    def _():
        acc_ref[...] = jnp.zeros_like(acc_ref)

    acc_ref[...] += lax.dot_general(
        a_ref[...], w_ref[...], _DN_T, preferred_element_type=jnp.float32)

    @pl.when(ki == nk - 1)
    def _():
        o_ref[...] = acc_ref[...]


def _qkv_call(x, Wq, Wk, Wv, interpret=False):
    BS, KC = 256, 512
    ns, nk = SEQ // BS, HIDDEN // KC
    hm = jnp.asarray(_HM)
    inv = jnp.asarray(_INV_FREQ)
    out_shapes = (
        jax.ShapeDtypeStruct((SEQ, DQ), jnp.float32),
        jax.ShapeDtypeStruct((SEQ, DKV), jnp.float32),
        jax.ShapeDtypeStruct((SEQ, DKV), jnp.float32),
        jax.ShapeDtypeStruct((SEQ, DKV), jnp.float32),
    )
    return pl.pallas_call(
        functools.partial(_qkv_kernel, nk=nk, bs=BS),
        grid=(ns, nk),
        in_specs=[
            pl.BlockSpec((BS, KC), lambda si, ki: (si, ki)),
            pl.BlockSpec((DQ, KC), lambda si, ki: (0, ki)),
            pl.BlockSpec((DKV, KC), lambda si, ki: (0, ki)),
            pl.BlockSpec((DKV, KC), lambda si, ki: (0, ki)),
            pl.BlockSpec((HD, HD), lambda si, ki: (0, 0)),
            pl.BlockSpec((1, HD // 2), lambda si, ki: (0, 0)),
        ],
        out_specs=(
            pl.BlockSpec((BS, DQ), lambda si, ki: (si, 0)),
            pl.BlockSpec((BS, DKV), lambda si, ki: (si, 0)),
            pl.BlockSpec((BS, DKV), lambda si, ki: (si, 0)),
            pl.BlockSpec((BS, DKV), lambda si, ki: (si, 0)),
        ),
        out_shape=out_shapes,
        scratch_shapes=[pltpu.VMEM((BS, DQ + 2 * DKV), jnp.float32)],
        compiler_params=pltpu.CompilerParams(
            dimension_semantics=("arbitrary", "arbitrary")),
        interpret=interpret,
    )(x, Wq, Wk, Wv, hm, inv)


def _attn_call(q, k, v, interpret=False):
    BQ = BK = 256
    nq = SEQ // BQ
    groups = N_HEADS // N_KV
    return pl.pallas_call(
        functools.partial(_attn_kernel, bq=BQ, bk=BK),
        grid=(N_HEADS, nq),
        in_specs=[
            pl.BlockSpec((BQ, HD), lambda h, qi: (qi, h)),
            pl.BlockSpec((SEQ, HD), lambda h, qi: (0, h // groups)),
            pl.BlockSpec((SEQ, HD), lambda h, qi: (0, h // groups)),
        ],
        out_specs=pl.BlockSpec((BQ, HD), lambda h, qi: (qi, h)),
        out_shape=jax.ShapeDtypeStruct((SEQ, DQ), jnp.float32),
        compiler_params=pltpu.CompilerParams(
            dimension_semantics=("arbitrary", "arbitrary")),
        interpret=interpret,
    )(q, k, v)


def _oproj_call(a, Wo, interpret=False):
    BS, KC = 256, 512
    ns, nk = SEQ // BS, DQ // KC
    return pl.pallas_call(
        functools.partial(_oproj_kernel, nk=nk),
        grid=(ns, nk),
        in_specs=[
            pl.BlockSpec((BS, KC), lambda si, ki: (si, ki)),
            pl.BlockSpec((HIDDEN, KC), lambda si, ki: (0, ki)),
        ],
        out_specs=pl.BlockSpec((BS, HIDDEN), lambda si, ki: (si, 0)),
        out_shape=jax.ShapeDtypeStruct((SEQ, HIDDEN), jnp.float32),
        scratch_shapes=[pltpu.VMEM((BS, HIDDEN), jnp.float32)],
        compiler_params=pltpu.CompilerParams(
            dimension_semantics=("arbitrary", "arbitrary")),
        interpret=interpret,
    )(a, Wo)


def kernel(hidden_states, position_ids, Wq, Wk, Wv, Wo, interpret=False):
    x = hidden_states[0]
    q, k, v, had = _qkv_call(x, Wq, Wk, Wv, interpret=interpret)
    attn = _attn_call(q, k, v, interpret=interpret)
    out = _oproj_call(attn, Wo, interpret=interpret)
    return out[None], had.reshape(SEQ, N_KV, HD)


# speedup vs baseline: 1.1828x; 1.1828x over previous
import functools
import math

import jax
import jax.numpy as jnp
import numpy as np
from jax import lax
from jax.experimental import pallas as pl
from jax.experimental.pallas import tpu as pltpu

HIDDEN = 4096
N_HEADS = 32
N_KV = 8
HD = 128
SEQ = 2048
DQ = N_HEADS * HD
DKV = N_KV * HD
ROPE_THETA = 500000.0
SCALE = 1.0 / math.sqrt(HD)

_INV_FREQ = (1.0 / (ROPE_THETA ** (np.arange(0, HD, 2, dtype=np.float32) / HD))
             ).reshape(1, HD // 2)


def _hadamard_matrix(n):
    H = np.array([[1.0]], dtype=np.float32)
    while H.shape[0] < n:
        H = np.block([[H, H], [H, -H]]).astype(np.float32)
    return H

_HM = _hadamard_matrix(HD)

_DN_T = (((1,), (1,)), ((), ()))
_DN_N = (((1,), (0,)), ((), ()))


def _rope(x, cos, sin, heads):
    half = HD // 2
    parts = []
    for h in range(heads):
        b = h * HD
        x1 = x[:, b:b + half]
        x2 = x[:, b + half:b + HD]
        parts.append(x1 * cos - x2 * sin)
        parts.append(x2 * cos + x1 * sin)
    return jnp.concatenate(parts, axis=1)


def _cos_sin(if_ref):
    pos = lax.broadcasted_iota(jnp.int32, (SEQ, 1), 0).astype(jnp.float32)
    freqs = pos * if_ref[...]
    return jnp.cos(freqs), jnp.sin(freqs)


def _qproj_kernel(x_ref, w_ref, if_ref, q_ref, acc_ref, *, nk, heads):
    ki = pl.program_id(1)

    @pl.when(ki == 0)
    def _():
        acc_ref[...] = jnp.zeros_like(acc_ref)

    acc_ref[...] += lax.dot_general(
        x_ref[...], w_ref[...].astype(jnp.bfloat16), _DN_T,
        preferred_element_type=jnp.float32)

    @pl.when(ki == nk - 1)
    def _():
        cos, sin = _cos_sin(if_ref)
        q_ref[...] = _rope(acc_ref[...], cos, sin, heads).astype(jnp.bfloat16)


def _kvproj_kernel(x_ref, wk_ref, wv_ref, hm_ref, if_ref,
                   k_ref, v_ref, h_ref, acck_ref, accv_ref, *, nk):
    ki = pl.program_id(0)

    @pl.when(ki == 0)
    def _():
        acck_ref[...] = jnp.zeros_like(acck_ref)
        accv_ref[...] = jnp.zeros_like(accv_ref)

    x = x_ref[...]
    acck_ref[...] += lax.dot_general(
        x, wk_ref[...].astype(jnp.bfloat16), _DN_T,
        preferred_element_type=jnp.float32)
    accv_ref[...] += lax.dot_general(
        x, wv_ref[...].astype(jnp.bfloat16), _DN_T,
        preferred_element_type=jnp.float32)

    @pl.when(ki == nk - 1)
    def _():
        cos, sin = _cos_sin(if_ref)
        kr = _rope(acck_ref[...], cos, sin, N_KV)
        k_ref[...] = kr.astype(jnp.bfloat16)
        v_ref[...] = accv_ref[...].astype(jnp.bfloat16)
        hm = hm_ref[...]
        for h in range(N_KV):
            b = h * HD
            h_ref[:, b:b + HD] = lax.dot_general(
                kr[:, b:b + HD], hm, _DN_N, preferred_element_type=jnp.float32)


def _attn_kernel(q_ref, k_ref, v_ref, o_ref, *, bq, bk):
    qi = pl.program_id(1)
    q = q_ref[...]
    nkb = (qi + 1) * (bq // bk)
    neg = jnp.finfo(jnp.float32).min
    row = qi * bq + lax.broadcasted_iota(jnp.int32, (bq, bk), 0)

    def body(j, carry):
        m, l, acc = carry
        kb = k_ref[pl.ds(j * bk, bk), :]
        vb = v_ref[pl.ds(j * bk, bk), :]
        s = lax.dot_general(q, kb, _DN_T,
                            preferred_element_type=jnp.float32) * SCALE
        col = j * bk + lax.broadcasted_iota(jnp.int32, (bq, bk), 1)
        s = jnp.where(col <= row, s, neg)
        m2 = jnp.maximum(m, jnp.max(s, axis=1, keepdims=True))
        alpha = jnp.exp(m - m2)
        p = jnp.exp(s - m2)
        l2 = l * alpha + jnp.sum(p, axis=1, keepdims=True)
        acc2 = acc * alpha + lax.dot_general(
            p.astype(jnp.bfloat16), vb, _DN_N,
            preferred_element_type=jnp.float32)
        return m2, l2, acc2

    m0 = jnp.full((bq, 1), neg, jnp.float32)
    l0 = jnp.zeros((bq, 1), jnp.float32)
    a0 = jnp.zeros((bq, HD), jnp.float32)
    m, l, acc = lax.fori_loop(0, nkb, body, (m0, l0, a0))
    o_ref[...] = (acc / l).astype(jnp.bfloat16)


def _oproj_kernel(a_ref, w_ref, o_ref, acc_ref, *, nk):
    ki = pl.program_id(1)

    @pl.when(ki == 0)
    def _():
        acc_ref[...] = jnp.zeros_like(acc_ref)

    acc_ref[...] += lax.dot_general(
        a_ref[...], w_ref[...].astype(jnp.bfloat16), _DN_T,
        preferred_element_type=jnp.float32)

    @pl.when(ki == nk - 1)
    def _():
        o_ref[...] = acc_ref[...]


def _qproj_call(xb, Wq, interpret=False):
    NC, KC = 2048, 512
    nn, nk = DQ // NC, HIDDEN // KC
    inv = jnp.asarray(_INV_FREQ)
    return pl.pallas_call(
        functools.partial(_qproj_kernel, nk=nk, heads=NC // HD),
        grid=(nn, nk),
        in_specs=[
            pl.BlockSpec((SEQ, KC), lambda ni, ki: (0, ki)),
            pl.BlockSpec((NC, KC), lambda ni, ki: (ni, ki)),
            pl.BlockSpec((1, HD // 2), lambda ni, ki: (0, 0)),
        ],
        out_specs=pl.BlockSpec((SEQ, NC), lambda ni, ki: (0, ni)),
        out_shape=jax.ShapeDtypeStruct((SEQ, DQ), jnp.bfloat16),
        scratch_shapes=[pltpu.VMEM((SEQ, NC), jnp.float32)],
        compiler_params=pltpu.CompilerParams(
            dimension_semantics=("arbitrary", "arbitrary")),
        interpret=interpret,
    )(xb, Wq, inv)


def _kvproj_call(xb, Wk, Wv, interpret=False):
    KC = 512
    nk = HIDDEN // KC
    hm = jnp.asarray(_HM)
    inv = jnp.asarray(_INV_FREQ)
    out_shapes = (
        jax.ShapeDtypeStruct((SEQ, DKV), jnp.bfloat16),
        jax.ShapeDtypeStruct((SEQ, DKV), jnp.bfloat16),
        jax.ShapeDtypeStruct((SEQ, DKV), jnp.float32),
    )
    return pl.pallas_call(
        functools.partial(_kvproj_kernel, nk=nk),
        grid=(nk,),
        in_specs=[
            pl.BlockSpec((SEQ, KC), lambda ki: (0, ki)),
            pl.BlockSpec((DKV, KC), lambda ki: (0, ki)),
            pl.BlockSpec((DKV, KC), lambda ki: (0, ki)),
            pl.BlockSpec((HD, HD), lambda ki: (0, 0)),
            pl.BlockSpec((1, HD // 2), lambda ki: (0, 0)),
        ],
        out_specs=(
            pl.BlockSpec((SEQ, DKV), lambda ki: (0, 0)),
            pl.BlockSpec((SEQ, DKV), lambda ki: (0, 0)),
            pl.BlockSpec((SEQ, DKV), lambda ki: (0, 0)),
        ),
        out_shape=out_shapes,
        scratch_shapes=[pltpu.VMEM((SEQ, DKV), jnp.float32),
                        pltpu.VMEM((SEQ, DKV), jnp.float32)],
        compiler_params=pltpu.CompilerParams(
            dimension_semantics=("arbitrary",)),
        interpret=interpret,
    )(xb, Wk, Wv, hm, inv)


def _attn_call(q, k, v, interpret=False):
    BQ = BK = 256
    nq = SEQ // BQ
    groups = N_HEADS // N_KV
    return pl.pallas_call(
        functools.partial(_attn_kernel, bq=BQ, bk=BK),
        grid=(N_HEADS, nq),
        in_specs=[
            pl.BlockSpec((BQ, HD), lambda h, qi: (qi, h)),
            pl.BlockSpec((SEQ, HD), lambda h, qi: (0, h // groups)),
            pl.BlockSpec((SEQ, HD), lambda h, qi: (0, h // groups)),
        ],
        out_specs=pl.BlockSpec((BQ, HD), lambda h, qi: (qi, h)),
        out_shape=jax.ShapeDtypeStruct((SEQ, DQ), jnp.bfloat16),
        compiler_params=pltpu.CompilerParams(
            dimension_semantics=("arbitrary", "arbitrary")),
        interpret=interpret,
    )(q, k, v)


def _oproj_call(a, Wo, interpret=False):
    NC, KC = 1024, 512
    nn, nk = HIDDEN // NC, DQ // KC
    return pl.pallas_call(
        functools.partial(_oproj_kernel, nk=nk),
        grid=(nn, nk),
        in_specs=[
            pl.BlockSpec((SEQ, KC), lambda ni, ki: (0, ki)),
            pl.BlockSpec((NC, KC), lambda ni, ki: (ni, ki)),
        ],
        out_specs=pl.BlockSpec((SEQ, NC), lambda ni, ki: (0, ni)),
        out_shape=jax.ShapeDtypeStruct((SEQ, HIDDEN), jnp.float32),
        scratch_shapes=[pltpu.VMEM((SEQ, NC), jnp.float32)],
        compiler_params=pltpu.CompilerParams(
            dimension_semantics=("arbitrary", "arbitrary")),
        interpret=interpret,
    )(a, Wo)


def kernel(hidden_states, position_ids, Wq, Wk, Wv, Wo, interpret=False):
    xb = hidden_states[0].astype(jnp.bfloat16)
    q = _qproj_call(xb, Wq, interpret=interpret)
    k, v, had = _kvproj_call(xb, Wk, Wv, interpret=interpret)
    attn = _attn_call(q, k, v, interpret=interpret)
    out = _oproj_call(attn, Wo, interpret=interpret)
    return out[None], had.reshape(SEQ, N_KV, HD)


# speedup vs baseline: 1.8381x; 1.5541x over previous
import functools
import math

import jax
import jax.numpy as jnp
import numpy as np
from jax import lax
from jax.experimental import pallas as pl
from jax.experimental.pallas import tpu as pltpu

HIDDEN = 4096
N_HEADS = 32
N_KV = 8
HD = 128
SEQ = 2048
DQ = N_HEADS * HD
DKV = N_KV * HD
ROPE_THETA = 500000.0
SCALE = 1.0 / math.sqrt(HD)

_INV_FREQ = (1.0 / (ROPE_THETA ** (np.arange(0, HD, 2, dtype=np.float32) / HD))
             ).reshape(1, HD // 2)


def _hadamard_matrix(n):
    H = np.array([[1.0]], dtype=np.float32)
    while H.shape[0] < n:
        H = np.block([[H, H], [H, -H]]).astype(np.float32)
    return H

_HM = _hadamard_matrix(HD)

_DN_T = (((1,), (1,)), ((), ()))
_DN_N = (((1,), (0,)), ((), ()))


def _rope(x, cos, sin, heads):
    half = HD // 2
    parts = []
    for h in range(heads):
        b = h * HD
        x1 = x[:, b:b + half]
        x2 = x[:, b + half:b + HD]
        parts.append(x1 * cos - x2 * sin)
        parts.append(x2 * cos + x1 * sin)
    return jnp.concatenate(parts, axis=1)


def _cos_sin(if_ref):
    pos = lax.broadcasted_iota(jnp.int32, (SEQ, 1), 0).astype(jnp.float32)
    freqs = pos * if_ref[...]
    return jnp.cos(freqs), jnp.sin(freqs)


def _qproj_kernel(x_ref, w_ref, if_ref, q_ref, acc_ref, *, nk, heads):
    ki = pl.program_id(1)

    @pl.when(ki == 0)
    def _():
        acc_ref[...] = jnp.zeros_like(acc_ref)

    acc_ref[...] += lax.dot_general(
        x_ref[...], w_ref[...].astype(jnp.bfloat16), _DN_T,
        preferred_element_type=jnp.float32)

    @pl.when(ki == nk - 1)
    def _():
        cos, sin = _cos_sin(if_ref)
        q_ref[...] = _rope(acc_ref[...], cos, sin, heads).astype(jnp.bfloat16)


def _kvproj_kernel(x_ref, wk_ref, wv_ref, hm_ref, if_ref,
                   k_ref, v_ref, h_ref, acck_ref, accv_ref, *, nk):
    ki = pl.program_id(0)

    @pl.when(ki == 0)
    def _():
        acck_ref[...] = jnp.zeros_like(acck_ref)
        accv_ref[...] = jnp.zeros_like(accv_ref)

    x = x_ref[...]
    acck_ref[...] += lax.dot_general(
        x, wk_ref[...].astype(jnp.bfloat16), _DN_T,
        preferred_element_type=jnp.float32)
    accv_ref[...] += lax.dot_general(
        x, wv_ref[...].astype(jnp.bfloat16), _DN_T,
        preferred_element_type=jnp.float32)

    @pl.when(ki == nk - 1)
    def _():
        cos, sin = _cos_sin(if_ref)
        kr = _rope(acck_ref[...], cos, sin, N_KV)
        k_ref[...] = kr.astype(jnp.bfloat16)
        v_ref[...] = accv_ref[...].astype(jnp.bfloat16)
        hm = hm_ref[...]
        for h in range(N_KV):
            b = h * HD
            h_ref[:, b:b + HD] = lax.dot_general(
                kr[:, b:b + HD], hm, _DN_N, preferred_element_type=jnp.float32)


def _attn_kernel(q_ref, k_ref, v_ref, o_ref, *, bq, bk, groups):
    qi = pl.program_id(1)
    mrows = groups * bq
    q = jnp.concatenate(
        [q_ref[:, h * HD:(h + 1) * HD] for h in range(groups)], axis=0)
    q = (q.astype(jnp.float32) * SCALE).astype(jnp.bfloat16)
    neg = jnp.finfo(jnp.float32).min

    def step(j, carry, masked):
        m, l, acc = carry
        kb = k_ref[pl.ds(j * bk, bk), :]
        vb = v_ref[pl.ds(j * bk, bk), :]
        s = lax.dot_general(q, kb, _DN_T, preferred_element_type=jnp.float32)
        if masked:
            rowm = lax.broadcasted_iota(jnp.int32, (mrows, bk), 0) & (bq - 1)
            col = lax.broadcasted_iota(jnp.int32, (mrows, bk), 1)
            s = jnp.where(col <= rowm, s, neg)
        m2 = jnp.maximum(m, jnp.max(s, axis=1, keepdims=True))
        alpha = jnp.exp(m - m2)
        p = jnp.exp(s - m2)
        l2 = l * alpha + jnp.sum(p, axis=1, keepdims=True)
        acc2 = acc * alpha + lax.dot_general(
            p.astype(jnp.bfloat16), vb, _DN_N,
            preferred_element_type=jnp.float32)
        return m2, l2, acc2

    m0 = jnp.full((mrows, 1), neg, jnp.float32)
    l0 = jnp.zeros((mrows, 1), jnp.float32)
    a0 = jnp.zeros((mrows, HD), jnp.float32)
    nfull = qi * (bq // bk)
    carry = lax.fori_loop(0, nfull, lambda j, c: step(j, c, False),
                          (m0, l0, a0))
    m, l, acc = step(nfull, carry, True)
    o = acc / l
    for h in range(groups):
        o_ref[:, h * HD:(h + 1) * HD] = (
            o[h * bq:(h + 1) * bq, :].astype(jnp.bfloat16))


def _oproj_kernel(a_ref, w_ref, o_ref, acc_ref, *, nk):
    ki = pl.program_id(1)

    @pl.when(ki == 0)
    def _():
        acc_ref[...] = jnp.zeros_like(acc_ref)

    acc_ref[...] += lax.dot_general(
        a_ref[...], w_ref[...].astype(jnp.bfloat16), _DN_T,
        preferred_element_type=jnp.float32)

    @pl.when(ki == nk - 1)
    def _():
        o_ref[...] = acc_ref[...]


def _qproj_call(xb, Wq, interpret=False):
    NC, KC = 2048, 512
    nn, nk = DQ // NC, HIDDEN // KC
    inv = jnp.asarray(_INV_FREQ)
    return pl.pallas_call(
        functools.partial(_qproj_kernel, nk=nk, heads=NC // HD),
        grid=(nn, nk),
        in_specs=[
            pl.BlockSpec((SEQ, KC), lambda ni, ki: (0, ki)),
            pl.BlockSpec((NC, KC), lambda ni, ki: (ni, ki)),
            pl.BlockSpec((1, HD // 2), lambda ni, ki: (0, 0)),
        ],
        out_specs=pl.BlockSpec((SEQ, NC), lambda ni, ki: (0, ni)),
        out_shape=jax.ShapeDtypeStruct((SEQ, DQ), jnp.bfloat16),
        scratch_shapes=[pltpu.VMEM((SEQ, NC), jnp.float32)],
        compiler_params=pltpu.CompilerParams(
            dimension_semantics=("arbitrary", "arbitrary")),
        interpret=interpret,
    )(xb, Wq, inv)


def _kvproj_call(xb, Wk, Wv, interpret=False):
    KC = 512
    nk = HIDDEN // KC
    hm = jnp.asarray(_HM)
    inv = jnp.asarray(_INV_FREQ)
    out_shapes = (
        jax.ShapeDtypeStruct((SEQ, DKV), jnp.bfloat16),
        jax.ShapeDtypeStruct((SEQ, DKV), jnp.bfloat16),
        jax.ShapeDtypeStruct((SEQ, DKV), jnp.float32),
    )
    return pl.pallas_call(
        functools.partial(_kvproj_kernel, nk=nk),
        grid=(nk,),
        in_specs=[
            pl.BlockSpec((SEQ, KC), lambda ki: (0, ki)),
            pl.BlockSpec((DKV, KC), lambda ki: (0, ki)),
            pl.BlockSpec((DKV, KC), lambda ki: (0, ki)),
            pl.BlockSpec((HD, HD), lambda ki: (0, 0)),
            pl.BlockSpec((1, HD // 2), lambda ki: (0, 0)),
        ],
        out_specs=(
            pl.BlockSpec((SEQ, DKV), lambda ki: (0, 0)),
            pl.BlockSpec((SEQ, DKV), lambda ki: (0, 0)),
            pl.BlockSpec((SEQ, DKV), lambda ki: (0, 0)),
        ),
        out_shape=out_shapes,
        scratch_shapes=[pltpu.VMEM((SEQ, DKV), jnp.float32),
                        pltpu.VMEM((SEQ, DKV), jnp.float32)],
        compiler_params=pltpu.CompilerParams(
            dimension_semantics=("arbitrary",)),
        interpret=interpret,
    )(xb, Wk, Wv, hm, inv)


def _attn_call(q, k, v, interpret=False):
    BQ = BK = 256
    nq = SEQ // BQ
    groups = N_HEADS // N_KV
    GD = groups * HD
    return pl.pallas_call(
        functools.partial(_attn_kernel, bq=BQ, bk=BK, groups=groups),
        grid=(N_KV, nq),
        in_specs=[
            pl.BlockSpec((BQ, GD), lambda g, qi: (qi, g)),
            pl.BlockSpec((SEQ, HD), lambda g, qi: (0, g)),
            pl.BlockSpec((SEQ, HD), lambda g, qi: (0, g)),
        ],
        out_specs=pl.BlockSpec((BQ, GD), lambda g, qi: (qi, g)),
        out_shape=jax.ShapeDtypeStruct((SEQ, DQ), jnp.bfloat16),
        compiler_params=pltpu.CompilerParams(
            dimension_semantics=("arbitrary", "arbitrary")),
        interpret=interpret,
    )(q, k, v)


def _oproj_call(a, Wo, interpret=False):
    NC, KC = 1024, 512
    nn, nk = HIDDEN // NC, DQ // KC
    return pl.pallas_call(
        functools.partial(_oproj_kernel, nk=nk),
        grid=(nn, nk),
        in_specs=[
            pl.BlockSpec((SEQ, KC), lambda ni, ki: (0, ki)),
            pl.BlockSpec((NC, KC), lambda ni, ki: (ni, ki)),
        ],
        out_specs=pl.BlockSpec((SEQ, NC), lambda ni, ki: (0, ni)),
        out_shape=jax.ShapeDtypeStruct((SEQ, HIDDEN), jnp.float32),
        scratch_shapes=[pltpu.VMEM((SEQ, NC), jnp.float32)],
        compiler_params=pltpu.CompilerParams(
            dimension_semantics=("arbitrary", "arbitrary")),
        interpret=interpret,
    )(a, Wo)


def kernel(hidden_states, position_ids, Wq, Wk, Wv, Wo, interpret=False):
    xb = hidden_states[0].astype(jnp.bfloat16)
    q = _qproj_call(xb, Wq, interpret=interpret)
    k, v, had = _kvproj_call(xb, Wk, Wv, interpret=interpret)
    attn = _attn_call(q, k, v, interpret=interpret)
    out = _oproj_call(attn, Wo, interpret=interpret)
    return out[None], had.reshape(SEQ, N_KV, HD)


# speedup vs baseline: 2.1410x; 1.1648x over previous
import functools
import math

import jax
import jax.numpy as jnp
import numpy as np
from jax import lax
from jax.experimental import pallas as pl
from jax.experimental.pallas import tpu as pltpu

HIDDEN = 4096
N_HEADS = 32
N_KV = 8
HD = 128
SEQ = 2048
DQ = N_HEADS * HD
DKV = N_KV * HD
ROPE_THETA = 500000.0
SCALE = 1.0 / math.sqrt(HD)

_INV_FREQ = (1.0 / (ROPE_THETA ** (np.arange(0, HD, 2, dtype=np.float32) / HD))
             ).reshape(1, HD // 2)


def _hadamard_matrix(n):
    H = np.array([[1.0]], dtype=np.float32)
    while H.shape[0] < n:
        H = np.block([[H, H], [H, -H]]).astype(np.float32)
    return H

_HM = _hadamard_matrix(HD)

_DN_T = (((1,), (1,)), ((), ()))
_DN_N = (((1,), (0,)), ((), ()))


def _rope(x, cos, sin, heads):
    half = HD // 2
    parts = []
    for h in range(heads):
        b = h * HD
        x1 = x[:, b:b + half]
        x2 = x[:, b + half:b + HD]
        parts.append(x1 * cos - x2 * sin)
        parts.append(x2 * cos + x1 * sin)
    return jnp.concatenate(parts, axis=1)


def _cos_sin(if_ref):
    pos = lax.broadcasted_iota(jnp.int32, (SEQ, 1), 0).astype(jnp.float32)
    freqs = pos * if_ref[...]
    return jnp.cos(freqs), jnp.sin(freqs)


def _qproj_kernel(x_ref, w_ref, if_ref, q_ref, acc_ref, *, nk, heads):
    ki = pl.program_id(1)

    @pl.when(ki == 0)
    def _():
        acc_ref[...] = jnp.zeros_like(acc_ref)

    acc_ref[...] += lax.dot_general(
        x_ref[...], w_ref[...].astype(jnp.bfloat16), _DN_T,
        preferred_element_type=jnp.float32)

    @pl.when(ki == nk - 1)
    def _():
        cos, sin = _cos_sin(if_ref)
        q_ref[...] = _rope(acc_ref[...], cos, sin, heads).astype(jnp.bfloat16)


def _kvproj_kernel(x_ref, wk_ref, wv_ref, hm_ref, if_ref,
                   k_ref, v_ref, h_ref, acck_ref, accv_ref, *, nk):
    ki = pl.program_id(0)

    @pl.when(ki == 0)
    def _():
        acck_ref[...] = jnp.zeros_like(acck_ref)
        accv_ref[...] = jnp.zeros_like(accv_ref)

    x = x_ref[...]
    acck_ref[...] += lax.dot_general(
        x, wk_ref[...].astype(jnp.bfloat16), _DN_T,
        preferred_element_type=jnp.float32)
    accv_ref[...] += lax.dot_general(
        x, wv_ref[...].astype(jnp.bfloat16), _DN_T,
        preferred_element_type=jnp.float32)

    @pl.when(ki == nk - 1)
    def _():
        cos, sin = _cos_sin(if_ref)
        kr = _rope(acck_ref[...], cos, sin, N_KV)
        k_ref[...] = kr.astype(jnp.bfloat16)
        v_ref[...] = accv_ref[...].astype(jnp.bfloat16)
        hm = hm_ref[...]
        for h in range(N_KV):
            b = h * HD
            h_ref[:, b:b + HD] = lax.dot_general(
                kr[:, b:b + HD], hm, _DN_N, preferred_element_type=jnp.float32)


def _attn_kernel(q_ref, k_ref, v_ref, o_ref, *, bq, bk, groups):
    qi = pl.program_id(1)
    mrows = groups * bq
    q = jnp.concatenate(
        [q_ref[:, h * HD:(h + 1) * HD] for h in range(groups)], axis=0)
    q = (q.astype(jnp.float32) * (SCALE * math.log2(math.e))
         ).astype(jnp.bfloat16)
    neg = jnp.finfo(jnp.float32).min

    def step(j, carry, masked):
        m, l, acc = carry
        kb = k_ref[pl.ds(j * bk, bk), :]
        vb = v_ref[pl.ds(j * bk, bk), :]
        s = lax.dot_general(q, kb, _DN_T, preferred_element_type=jnp.float32)
        if masked:
            rowm = lax.broadcasted_iota(jnp.int32, (mrows, bk), 0) & (bq - 1)
            col = lax.broadcasted_iota(jnp.int32, (mrows, bk), 1)
            s = jnp.where(col <= rowm, s, neg)
        m2 = jnp.maximum(m, jnp.max(s, axis=1, keepdims=True))
        alpha = jnp.exp2(m - m2)
        p = jnp.exp2(s - m2)
        l2 = l * alpha + jnp.sum(p, axis=1, keepdims=True)
        acc2 = acc * alpha + lax.dot_general(
            p.astype(jnp.bfloat16), vb, _DN_N,
            preferred_element_type=jnp.float32)
        return m2, l2, acc2

    m0 = jnp.full((mrows, 1), neg, jnp.float32)
    l0 = jnp.zeros((mrows, 1), jnp.float32)
    a0 = jnp.zeros((mrows, HD), jnp.float32)
    nfull = qi * (bq // bk)
    carry = lax.fori_loop(0, nfull, lambda j, c: step(j, c, False),
                          (m0, l0, a0))
    m, l, acc = step(nfull, carry, True)
    o = acc / l
    for h in range(groups):
        o_ref[:, h * HD:(h + 1) * HD] = (
            o[h * bq:(h + 1) * bq, :].astype(jnp.bfloat16))


def _oproj_kernel(a_ref, w_ref, o_ref, acc_ref, *, nk):
    ki = pl.program_id(1)

    @pl.when(ki == 0)
    def _():
        acc_ref[...] = jnp.zeros_like(acc_ref)

    acc_ref[...] += lax.dot_general(
        a_ref[...], w_ref[...].astype(jnp.bfloat16), _DN_T,
        preferred_element_type=jnp.float32)

    @pl.when(ki == nk - 1)
    def _():
        o_ref[...] = acc_ref[...]


def _qproj_call(xb, Wq, interpret=False):
    NC, KC = 2048, 512
    nn, nk = DQ // NC, HIDDEN // KC
    inv = jnp.asarray(_INV_FREQ)
    return pl.pallas_call(
        functools.partial(_qproj_kernel, nk=nk, heads=NC // HD),
        grid=(nn, nk),
        in_specs=[
            pl.BlockSpec((SEQ, KC), lambda ni, ki: (0, ki)),
            pl.BlockSpec((NC, KC), lambda ni, ki: (ni, ki)),
            pl.BlockSpec((1, HD // 2), lambda ni, ki: (0, 0)),
        ],
        out_specs=pl.BlockSpec((SEQ, NC), lambda ni, ki: (0, ni)),
        out_shape=jax.ShapeDtypeStruct((SEQ, DQ), jnp.bfloat16),
        scratch_shapes=[pltpu.VMEM((SEQ, NC), jnp.float32)],
        compiler_params=pltpu.CompilerParams(
            dimension_semantics=("arbitrary", "arbitrary")),
        interpret=interpret,
    )(xb, Wq, inv)


def _kvproj_call(xb, Wk, Wv, interpret=False):
    KC = 512
    nk = HIDDEN // KC
    hm = jnp.asarray(_HM)
    inv = jnp.asarray(_INV_FREQ)
    out_shapes = (
        jax.ShapeDtypeStruct((SEQ, DKV), jnp.bfloat16),
        jax.ShapeDtypeStruct((SEQ, DKV), jnp.bfloat16),
        jax.ShapeDtypeStruct((SEQ, DKV), jnp.float32),
    )
    return pl.pallas_call(
        functools.partial(_kvproj_kernel, nk=nk),
        grid=(nk,),
        in_specs=[
            pl.BlockSpec((SEQ, KC), lambda ki: (0, ki)),
            pl.BlockSpec((DKV, KC), lambda ki: (0, ki)),
            pl.BlockSpec((DKV, KC), lambda ki: (0, ki)),
            pl.BlockSpec((HD, HD), lambda ki: (0, 0)),
            pl.BlockSpec((1, HD // 2), lambda ki: (0, 0)),
        ],
        out_specs=(
            pl.BlockSpec((SEQ, DKV), lambda ki: (0, 0)),
            pl.BlockSpec((SEQ, DKV), lambda ki: (0, 0)),
            pl.BlockSpec((SEQ, DKV), lambda ki: (0, 0)),
        ),
        out_shape=out_shapes,
        scratch_shapes=[pltpu.VMEM((SEQ, DKV), jnp.float32),
                        pltpu.VMEM((SEQ, DKV), jnp.float32)],
        compiler_params=pltpu.CompilerParams(
            dimension_semantics=("arbitrary",)),
        interpret=interpret,
    )(xb, Wk, Wv, hm, inv)


def _attn_call(q, k, v, interpret=False):
    BQ = BK = 512
    nq = SEQ // BQ
    groups = N_HEADS // N_KV
    GD = groups * HD
    return pl.pallas_call(
        functools.partial(_attn_kernel, bq=BQ, bk=BK, groups=groups),
        grid=(N_KV, nq),
        in_specs=[
            pl.BlockSpec((BQ, GD), lambda g, qi: (qi, g)),
            pl.BlockSpec((SEQ, HD), lambda g, qi: (0, g)),
            pl.BlockSpec((SEQ, HD), lambda g, qi: (0, g)),
        ],
        out_specs=pl.BlockSpec((BQ, GD), lambda g, qi: (qi, g)),
        out_shape=jax.ShapeDtypeStruct((SEQ, DQ), jnp.bfloat16),
        compiler_params=pltpu.CompilerParams(
            dimension_semantics=("arbitrary", "arbitrary")),
        interpret=interpret,
    )(q, k, v)


def _oproj_call(a, Wo, interpret=False):
    NC, KC = 1024, 512
    nn, nk = HIDDEN // NC, DQ // KC
    return pl.pallas_call(
        functools.partial(_oproj_kernel, nk=nk),
        grid=(nn, nk),
        in_specs=[
            pl.BlockSpec((SEQ, KC), lambda ni, ki: (0, ki)),
            pl.BlockSpec((NC, KC), lambda ni, ki: (ni, ki)),
        ],
        out_specs=pl.BlockSpec((SEQ, NC), lambda ni, ki: (0, ni)),
        out_shape=jax.ShapeDtypeStruct((SEQ, HIDDEN), jnp.float32),
        scratch_shapes=[pltpu.VMEM((SEQ, NC), jnp.float32)],
        compiler_params=pltpu.CompilerParams(
            dimension_semantics=("arbitrary", "arbitrary")),
        interpret=interpret,
    )(a, Wo)


def kernel(hidden_states, position_ids, Wq, Wk, Wv, Wo, interpret=False):
    xb = hidden_states[0].astype(jnp.bfloat16)
    q = _qproj_call(xb, Wq, interpret=interpret)
    k, v, had = _kvproj_call(xb, Wk, Wv, interpret=interpret)
    attn = _attn_call(q, k, v, interpret=interpret)
    out = _oproj_call(attn, Wo, interpret=interpret)
    return out[None], had.reshape(SEQ, N_KV, HD)


# speedup vs baseline: 2.3414x; 1.0936x over previous
import functools
import math

import jax
import jax.numpy as jnp
import numpy as np
from jax import lax
from jax.experimental import pallas as pl
from jax.experimental.pallas import tpu as pltpu

HIDDEN = 4096
N_HEADS = 32
N_KV = 8
HD = 128
SEQ = 2048
DQ = N_HEADS * HD
DKV = N_KV * HD
ROPE_THETA = 500000.0
SCALE = 1.0 / math.sqrt(HD)

_INV_FREQ2 = np.tile(
    1.0 / (ROPE_THETA ** (np.arange(0, HD, 2, dtype=np.float32) / HD)),
    2).reshape(1, HD)
_SIGN = np.concatenate([-np.ones(HD // 2, np.float32),
                        np.ones(HD // 2, np.float32)]).reshape(1, HD)


def _hadamard_matrix(n):
    H = np.array([[1.0]], dtype=np.float32)
    while H.shape[0] < n:
        H = np.block([[H, H], [H, -H]]).astype(np.float32)
    return H

_HM = _hadamard_matrix(HD)

_DN_T = (((1,), (1,)), ((), ()))
_DN_N = (((1,), (0,)), ((), ()))


def _tables_kernel(if_ref, sg_ref, cos_ref, ssin_ref):
    pos = lax.broadcasted_iota(jnp.int32, (SEQ, 1), 0).astype(jnp.float32)
    f = pos * if_ref[...]
    cos_ref[...] = jnp.cos(f)
    ssin_ref[...] = sg_ref[...] * jnp.sin(f)


def _rope_head(x, cos, ssin):
    return x * cos + pltpu.roll(x, HD // 2, 1) * ssin


def _qproj_kernel(x_ref, w_ref, cos_ref, ssin_ref, q_ref, acc_ref, *, nk, nc):
    ki = pl.program_id(1)

    def dot():
        return lax.dot_general(
            x_ref[...], w_ref[...].astype(jnp.bfloat16), _DN_T,
            preferred_element_type=jnp.float32)

    @pl.when(ki == 0)
    def _():
        acc_ref[...] = dot()

    @pl.when(ki > 0)
    def _():
        acc_ref[...] += dot()

    @pl.when(ki == nk - 1)
    def _():
        acc = acc_ref[...]
        cos = cos_ref[...]
        ssin = ssin_ref[...]
        for h in range(nc // HD):
            b = h * HD
            q_ref[:, b:b + HD] = _rope_head(
                acc[:, b:b + HD], cos, ssin).astype(jnp.bfloat16)


def _kvproj_kernel(x_ref, wk_ref, wv_ref, cos_ref, ssin_ref,
                   k_ref, v_ref, acck_ref, accv_ref, *, nk):
    ki = pl.program_id(0)

    def dotk():
        return lax.dot_general(
            x_ref[...], wk_ref[...].astype(jnp.bfloat16), _DN_T,
            preferred_element_type=jnp.float32)

    def dotv():
        return lax.dot_general(
            x_ref[...], wv_ref[...].astype(jnp.bfloat16), _DN_T,
            preferred_element_type=jnp.float32)

    @pl.when(ki == 0)
    def _():
        acck_ref[...] = dotk()
        accv_ref[...] = dotv()

    @pl.when(ki > 0)
    def _():
        acck_ref[...] += dotk()
        accv_ref[...] += dotv()

    @pl.when(ki == nk - 1)
    def _():
        acck = acck_ref[...]
        cos = cos_ref[...]
        ssin = ssin_ref[...]
        for h in range(N_KV):
            b = h * HD
            k_ref[:, b:b + HD] = _rope_head(
                acck[:, b:b + HD], cos, ssin).astype(jnp.bfloat16)
        v_ref[...] = accv_ref[...].astype(jnp.bfloat16)


def _hadamard_kernel(k_ref, hm_ref, h_ref):
    hm = hm_ref[...]
    for h in range(N_KV):
        b = h * HD
        h_ref[:, b:b + HD] = lax.dot_general(
            k_ref[:, b:b + HD], hm, _DN_N, preferred_element_type=jnp.float32)


def _attn_kernel(q_ref, k_ref, v_ref, o_ref, *, bq, bk, groups):
    qi = pl.program_id(1)
    mrows = groups * bq
    q = jnp.concatenate(
        [q_ref[:, h * HD:(h + 1) * HD] for h in range(groups)], axis=0)
    q = (q.astype(jnp.float32) * (SCALE * math.log2(math.e))
         ).astype(jnp.bfloat16)
    neg = jnp.finfo(jnp.float32).min

    def step(j, carry, masked):
        m, l, acc = carry
        kb = k_ref[pl.ds(j * bk, bk), :]
        vb = v_ref[pl.ds(j * bk, bk), :]
        s = lax.dot_general(q, kb, _DN_T, preferred_element_type=jnp.float32)
        if masked:
            rowm = lax.broadcasted_iota(jnp.int32, (mrows, bk), 0) & (bq - 1)
            col = lax.broadcasted_iota(jnp.int32, (mrows, bk), 1)
            s = jnp.where(col <= rowm, s, neg)
        m2 = jnp.maximum(m, jnp.max(s, axis=1, keepdims=True))
        alpha = jnp.exp2(m - m2)
        p = jnp.exp2(s - m2)
        l2 = l * alpha + jnp.sum(p, axis=1, keepdims=True)
        acc2 = acc * alpha + lax.dot_general(
            p.astype(jnp.bfloat16), vb, _DN_N,
            preferred_element_type=jnp.float32)
        return m2, l2, acc2

    m0 = jnp.full((mrows, 1), neg, jnp.float32)
    l0 = jnp.zeros((mrows, 1), jnp.float32)
    a0 = jnp.zeros((mrows, HD), jnp.float32)
    nfull = qi * (bq // bk)
    carry = lax.fori_loop(0, nfull, lambda j, c: step(j, c, False),
                          (m0, l0, a0))
    m, l, acc = step(nfull, carry, True)
    o = acc / l
    for h in range(groups):
        o_ref[:, h * HD:(h + 1) * HD] = (
            o[h * bq:(h + 1) * bq, :].astype(jnp.bfloat16))


def _oproj_kernel(a_ref, w_ref, o_ref, acc_ref, *, nk):
    ki = pl.program_id(1)

    def dot():
        return lax.dot_general(
            a_ref[...], w_ref[...].astype(jnp.bfloat16), _DN_T,
            preferred_element_type=jnp.float32)

    @pl.when(ki == 0)
    def _():
        acc_ref[...] = dot()

    @pl.when(ki > 0)
    def _():
        acc_ref[...] += dot()

    @pl.when(ki == nk - 1)
    def _():
        o_ref[...] = acc_ref[...]


def _tables_call(interpret=False):
    invf = jnp.asarray(_INV_FREQ2)
    sign = jnp.asarray(_SIGN)
    return pl.pallas_call(
        _tables_kernel,
        out_shape=(jax.ShapeDtypeStruct((SEQ, HD), jnp.float32),
                   jax.ShapeDtypeStruct((SEQ, HD), jnp.float32)),
        interpret=interpret,
    )(invf, sign)


def _qproj_call(xb, Wq, cos, ssin, interpret=False):
    NC, KC = 1024, 1024
    nn, nk = DQ // NC, HIDDEN // KC
    return pl.pallas_call(
        functools.partial(_qproj_kernel, nk=nk, nc=NC),
        grid=(nn, nk),
        in_specs=[
            pl.BlockSpec((SEQ, KC), lambda ni, ki: (0, ki)),
            pl.BlockSpec((NC, KC), lambda ni, ki: (ni, ki)),
            pl.BlockSpec((SEQ, HD), lambda ni, ki: (0, 0)),
            pl.BlockSpec((SEQ, HD), lambda ni, ki: (0, 0)),
        ],
        out_specs=pl.BlockSpec((SEQ, NC), lambda ni, ki: (0, ni)),
        out_shape=jax.ShapeDtypeStruct((SEQ, DQ), jnp.bfloat16),
        scratch_shapes=[pltpu.VMEM((SEQ, NC), jnp.float32)],
        compiler_params=pltpu.CompilerParams(
            dimension_semantics=("arbitrary", "arbitrary")),
        interpret=interpret,
    )(xb, Wq, cos, ssin)


def _kvproj_call(xb, Wk, Wv, cos, ssin, interpret=False):
    KC = 512
    nk = HIDDEN // KC
    out_shapes = (
        jax.ShapeDtypeStruct((SEQ, DKV), jnp.bfloat16),
        jax.ShapeDtypeStruct((SEQ, DKV), jnp.bfloat16),
    )
    return pl.pallas_call(
        functools.partial(_kvproj_kernel, nk=nk),
        grid=(nk,),
        in_specs=[
            pl.BlockSpec((SEQ, KC), lambda ki: (0, ki)),
            pl.BlockSpec((DKV, KC), lambda ki: (0, ki)),
            pl.BlockSpec((DKV, KC), lambda ki: (0, ki)),
            pl.BlockSpec((SEQ, HD), lambda ki: (0, 0)),
            pl.BlockSpec((SEQ, HD), lambda ki: (0, 0)),
        ],
        out_specs=(
            pl.BlockSpec((SEQ, DKV), lambda ki: (0, 0)),
            pl.BlockSpec((SEQ, DKV), lambda ki: (0, 0)),
        ),
        out_shape=out_shapes,
        scratch_shapes=[pltpu.VMEM((SEQ, DKV), jnp.float32),
                        pltpu.VMEM((SEQ, DKV), jnp.float32)],
        compiler_params=pltpu.CompilerParams(
            dimension_semantics=("arbitrary",)),
        interpret=interpret,
    )(xb, Wk, Wv, cos, ssin)


def _hadamard_call(k, interpret=False):
    BS = 512
    hm = jnp.asarray(_HM)
    return pl.pallas_call(
        _hadamard_kernel,
        grid=(SEQ // BS,),
        in_specs=[
            pl.BlockSpec((BS, DKV), lambda i: (i, 0)),
            pl.BlockSpec((HD, HD), lambda i: (0, 0)),
        ],
        out_specs=pl.BlockSpec((BS, DKV), lambda i: (i, 0)),
        out_shape=jax.ShapeDtypeStruct((SEQ, DKV), jnp.float32),
        compiler_params=pltpu.CompilerParams(
            dimension_semantics=("arbitrary",)),
        interpret=interpret,
    )(k, hm)


def _attn_call(q, k, v, interpret=False):
    BQ = BK = 512
    nq = SEQ // BQ
    groups = N_HEADS // N_KV
    GD = groups * HD
    return pl.pallas_call(
        functools.partial(_attn_kernel, bq=BQ, bk=BK, groups=groups),
        grid=(N_KV, nq),
        in_specs=[
            pl.BlockSpec((BQ, GD), lambda g, qi: (qi, g)),
            pl.BlockSpec((SEQ, HD), lambda g, qi: (0, g)),
            pl.BlockSpec((SEQ, HD), lambda g, qi: (0, g)),
        ],
        out_specs=pl.BlockSpec((BQ, GD), lambda g, qi: (qi, g)),
        out_shape=jax.ShapeDtypeStruct((SEQ, DQ), jnp.bfloat16),
        compiler_params=pltpu.CompilerParams(
            dimension_semantics=("arbitrary", "arbitrary")),
        interpret=interpret,
    )(q, k, v)


def _oproj_call(a, Wo, interpret=False):
    NC, KC = 1024, 1024
    nn, nk = HIDDEN // NC, DQ // KC
    return pl.pallas_call(
        functools.partial(_oproj_kernel, nk=nk),
        grid=(nn, nk),
        in_specs=[
            pl.BlockSpec((SEQ, KC), lambda ni, ki: (0, ki)),
            pl.BlockSpec((NC, KC), lambda ni, ki: (ni, ki)),
        ],
        out_specs=pl.BlockSpec((SEQ, NC), lambda ni, ki: (0, ni)),
        out_shape=jax.ShapeDtypeStruct((SEQ, HIDDEN), jnp.float32),
        scratch_shapes=[pltpu.VMEM((SEQ, NC), jnp.float32)],
        compiler_params=pltpu.CompilerParams(
            dimension_semantics=("arbitrary", "arbitrary")),
        interpret=interpret,
    )(a, Wo)


def kernel(hidden_states, position_ids, Wq, Wk, Wv, Wo, interpret=False):
    xb = hidden_states[0].astype(jnp.bfloat16)
    cos, ssin = _tables_call(interpret=interpret)
    q = _qproj_call(xb, Wq, cos, ssin, interpret=interpret)
    k, v = _kvproj_call(xb, Wk, Wv, cos, ssin, interpret=interpret)
    had = _hadamard_call(k, interpret=interpret)
    attn = _attn_call(q, k, v, interpret=interpret)
    out = _oproj_call(attn, Wo, interpret=interpret)
    return out[None], had.reshape(SEQ, N_KV, HD)


# speedup vs baseline: 2.3632x; 1.0093x over previous
import functools
import math

import jax
import jax.numpy as jnp
import numpy as np
from jax import lax
from jax.experimental import pallas as pl
from jax.experimental.pallas import tpu as pltpu

HIDDEN = 4096
N_HEADS = 32
N_KV = 8
HD = 128
SEQ = 2048
DQ = N_HEADS * HD
DKV = N_KV * HD
ROPE_THETA = 500000.0
SCALE = 1.0 / math.sqrt(HD)

_INV_FREQ2 = np.tile(
    1.0 / (ROPE_THETA ** (np.arange(0, HD, 2, dtype=np.float32) / HD)),
    2).reshape(1, HD)
_SIGN = np.concatenate([-np.ones(HD // 2, np.float32),
                        np.ones(HD // 2, np.float32)]).reshape(1, HD)


def _hadamard_matrix(n):
    H = np.array([[1.0]], dtype=np.float32)
    while H.shape[0] < n:
        H = np.block([[H, H], [H, -H]]).astype(np.float32)
    return H

_HM = _hadamard_matrix(HD)

_DN_T = (((1,), (1,)), ((), ()))
_DN_N = (((1,), (0,)), ((), ()))


def _tables_kernel(x_ref, if_ref, sg_ref, xb_ref, cos_ref, ssin_ref, *, bs):
    i = pl.program_id(0)
    xb_ref[...] = x_ref[...].astype(jnp.bfloat16)
    pos = (i * bs + lax.broadcasted_iota(jnp.int32, (bs, 1), 0)
           ).astype(jnp.float32)
    f = pos * if_ref[...]
    cos_ref[...] = jnp.cos(f)
    ssin_ref[...] = sg_ref[...] * jnp.sin(f)


def _rope_head(x, cos, ssin):
    return x * cos + pltpu.roll(x, HD // 2, 1) * ssin


def _qproj_kernel(x_ref, w_ref, cos_ref, ssin_ref, q_ref, acc_ref, *, nk, nc):
    ki = pl.program_id(1)

    def dot():
        return lax.dot_general(
            x_ref[...], w_ref[...].astype(jnp.bfloat16), _DN_T,
            preferred_element_type=jnp.float32)

    @pl.when(ki == 0)
    def _():
        acc_ref[...] = dot()

    @pl.when(ki > 0)
    def _():
        acc_ref[...] += dot()

    @pl.when(ki == nk - 1)
    def _():
        acc = acc_ref[...]
        cos = cos_ref[...]
        ssin = ssin_ref[...]
        for h in range(nc // HD):
            b = h * HD
            q_ref[:, b:b + HD] = _rope_head(
                acc[:, b:b + HD], cos, ssin).astype(jnp.bfloat16)


def _kvproj_kernel(x_ref, wk_ref, wv_ref, cos_ref, ssin_ref,
                   k_ref, v_ref, acck_ref, accv_ref, *, nk):
    ki = pl.program_id(0)

    def dotk():
        return lax.dot_general(
            x_ref[...], wk_ref[...].astype(jnp.bfloat16), _DN_T,
            preferred_element_type=jnp.float32)

    def dotv():
        return lax.dot_general(
            x_ref[...], wv_ref[...].astype(jnp.bfloat16), _DN_T,
            preferred_element_type=jnp.float32)

    @pl.when(ki == 0)
    def _():
        acck_ref[...] = dotk()
        accv_ref[...] = dotv()

    @pl.when(ki > 0)
    def _():
        acck_ref[...] += dotk()
        accv_ref[...] += dotv()

    @pl.when(ki == nk - 1)
    def _():
        acck = acck_ref[...]
        cos = cos_ref[...]
        ssin = ssin_ref[...]
        for h in range(N_KV):
            b = h * HD
            k_ref[:, b:b + HD] = _rope_head(
                acck[:, b:b + HD], cos, ssin).astype(jnp.bfloat16)
        v_ref[...] = accv_ref[...].astype(jnp.bfloat16)


def _hadamard_kernel(k_ref, hm_ref, h_ref):
    hm = hm_ref[...]
    for h in range(N_KV):
        b = h * HD
        h_ref[:, b:b + HD] = lax.dot_general(
            k_ref[:, b:b + HD], hm, _DN_N, preferred_element_type=jnp.float32)


def _attn_kernel(q_ref, k_ref, v_ref, o_ref, *, bq, bk, groups):
    qi = pl.program_id(1)
    mrows = groups * bq
    q = jnp.concatenate(
        [q_ref[:, h * HD:(h + 1) * HD] for h in range(groups)], axis=0)
    q = (q.astype(jnp.float32) * (SCALE * math.log2(math.e))
         ).astype(jnp.bfloat16)
    neg = jnp.finfo(jnp.float32).min

    def step(j, carry, masked):
        m, l, acc = carry
        kb = k_ref[pl.ds(j * bk, bk), :]
        vb = v_ref[pl.ds(j * bk, bk), :]
        s = lax.dot_general(q, kb, _DN_T, preferred_element_type=jnp.float32)
        if masked:
            rowm = lax.broadcasted_iota(jnp.int32, (mrows, bk), 0) & (bq - 1)
            col = lax.broadcasted_iota(jnp.int32, (mrows, bk), 1)
            s = jnp.where(col <= rowm, s, neg)
        m2 = jnp.maximum(m, jnp.max(s, axis=1, keepdims=True))
        alpha = jnp.exp2(m - m2)
        p = jnp.exp2(s - m2)
        l2 = l * alpha + jnp.sum(p, axis=1, keepdims=True)
        acc2 = acc * alpha + lax.dot_general(
            p.astype(jnp.bfloat16), vb, _DN_N,
            preferred_element_type=jnp.float32)
        return m2, l2, acc2

    m0 = jnp.full((mrows, 1), neg, jnp.float32)
    l0 = jnp.zeros((mrows, 1), jnp.float32)
    a0 = jnp.zeros((mrows, HD), jnp.float32)
    nfull = qi * (bq // bk)
    carry = lax.fori_loop(0, nfull, lambda j, c: step(j, c, False),
                          (m0, l0, a0))
    m, l, acc = step(nfull, carry, True)
    o = acc / l
    for h in range(groups):
        o_ref[:, h * HD:(h + 1) * HD] = (
            o[h * bq:(h + 1) * bq, :].astype(jnp.bfloat16))


def _oproj_kernel(a_ref, w_ref, o_ref, acc_ref, *, nk):
    ki = pl.program_id(1)

    def dot():
        return lax.dot_general(
            a_ref[...], w_ref[...].astype(jnp.bfloat16), _DN_T,
            preferred_element_type=jnp.float32)

    @pl.when(ki == 0)
    def _():
        acc_ref[...] = dot()

    @pl.when(ki > 0)
    def _():
        acc_ref[...] += dot()

    @pl.when(ki == nk - 1)
    def _():
        o_ref[...] = acc_ref[...]


def _tables_call(x, interpret=False):
    BS = 512
    invf = jnp.asarray(_INV_FREQ2)
    sign = jnp.asarray(_SIGN)
    return pl.pallas_call(
        functools.partial(_tables_kernel, bs=BS),
        grid=(SEQ // BS,),
        in_specs=[
            pl.BlockSpec((BS, HIDDEN), lambda i: (i, 0)),
            pl.BlockSpec((1, HD), lambda i: (0, 0)),
            pl.BlockSpec((1, HD), lambda i: (0, 0)),
        ],
        out_specs=(
            pl.BlockSpec((BS, HIDDEN), lambda i: (i, 0)),
            pl.BlockSpec((BS, HD), lambda i: (i, 0)),
            pl.BlockSpec((BS, HD), lambda i: (i, 0)),
        ),
        out_shape=(jax.ShapeDtypeStruct((SEQ, HIDDEN), jnp.bfloat16),
                   jax.ShapeDtypeStruct((SEQ, HD), jnp.float32),
                   jax.ShapeDtypeStruct((SEQ, HD), jnp.float32)),
        compiler_params=pltpu.CompilerParams(
            dimension_semantics=("arbitrary",)),
        interpret=interpret,
    )(x, invf, sign)


def _qproj_call(xb, Wq, cos, ssin, interpret=False):
    NC, KC = 1024, 1024
    nn, nk = DQ // NC, HIDDEN // KC
    return pl.pallas_call(
        functools.partial(_qproj_kernel, nk=nk, nc=NC),
        grid=(nn, nk),
        in_specs=[
            pl.BlockSpec((SEQ, KC), lambda ni, ki: (0, ki)),
            pl.BlockSpec((NC, KC), lambda ni, ki: (ni, ki)),
            pl.BlockSpec((SEQ, HD), lambda ni, ki: (0, 0)),
            pl.BlockSpec((SEQ, HD), lambda ni, ki: (0, 0)),
        ],
        out_specs=pl.BlockSpec((SEQ, NC), lambda ni, ki: (0, ni)),
        out_shape=jax.ShapeDtypeStruct((SEQ, DQ), jnp.bfloat16),
        scratch_shapes=[pltpu.VMEM((SEQ, NC), jnp.float32)],
        compiler_params=pltpu.CompilerParams(
            dimension_semantics=("arbitrary", "arbitrary")),
        interpret=interpret,
    )(xb, Wq, cos, ssin)


def _kvproj_call(xb, Wk, Wv, cos, ssin, interpret=False):
    KC = 512
    nk = HIDDEN // KC
    out_shapes = (
        jax.ShapeDtypeStruct((SEQ, DKV), jnp.bfloat16),
        jax.ShapeDtypeStruct((SEQ, DKV), jnp.bfloat16),
    )
    return pl.pallas_call(
        functools.partial(_kvproj_kernel, nk=nk),
        grid=(nk,),
        in_specs=[
            pl.BlockSpec((SEQ, KC), lambda ki: (0, ki)),
            pl.BlockSpec((DKV, KC), lambda ki: (0, ki)),
            pl.BlockSpec((DKV, KC), lambda ki: (0, ki)),
            pl.BlockSpec((SEQ, HD), lambda ki: (0, 0)),
            pl.BlockSpec((SEQ, HD), lambda ki: (0, 0)),
        ],
        out_specs=(
            pl.BlockSpec((SEQ, DKV), lambda ki: (0, 0)),
            pl.BlockSpec((SEQ, DKV), lambda ki: (0, 0)),
        ),
        out_shape=out_shapes,
        scratch_shapes=[pltpu.VMEM((SEQ, DKV), jnp.float32),
                        pltpu.VMEM((SEQ, DKV), jnp.float32)],
        compiler_params=pltpu.CompilerParams(
            dimension_semantics=("arbitrary",)),
        interpret=interpret,
    )(xb, Wk, Wv, cos, ssin)


def _hadamard_call(k, interpret=False):
    BS = 512
    hm = jnp.asarray(_HM)
    return pl.pallas_call(
        _hadamard_kernel,
        grid=(SEQ // BS,),
        in_specs=[
            pl.BlockSpec((BS, DKV), lambda i: (i, 0)),
            pl.BlockSpec((HD, HD), lambda i: (0, 0)),
        ],
        out_specs=pl.BlockSpec((BS, DKV), lambda i: (i, 0)),
        out_shape=jax.ShapeDtypeStruct((SEQ, DKV), jnp.float32),
        compiler_params=pltpu.CompilerParams(
            dimension_semantics=("arbitrary",)),
        interpret=interpret,
    )(k, hm)


def _attn_call(q, k, v, interpret=False):
    BQ = BK = 512
    nq = SEQ // BQ
    groups = N_HEADS // N_KV
    GD = groups * HD
    return pl.pallas_call(
        functools.partial(_attn_kernel, bq=BQ, bk=BK, groups=groups),
        grid=(N_KV, nq),
        in_specs=[
            pl.BlockSpec((BQ, GD), lambda g, qi: (qi, g)),
            pl.BlockSpec((SEQ, HD), lambda g, qi: (0, g)),
            pl.BlockSpec((SEQ, HD), lambda g, qi: (0, g)),
        ],
        out_specs=pl.BlockSpec((BQ, GD), lambda g, qi: (qi, g)),
        out_shape=jax.ShapeDtypeStruct((SEQ, DQ), jnp.bfloat16),
        compiler_params=pltpu.CompilerParams(
            dimension_semantics=("arbitrary", "arbitrary")),
        interpret=interpret,
    )(q, k, v)


def _oproj_call(a, Wo, interpret=False):
    NC, KC = 1024, 1024
    nn, nk = HIDDEN // NC, DQ // KC
    return pl.pallas_call(
        functools.partial(_oproj_kernel, nk=nk),
        grid=(nn, nk),
        in_specs=[
            pl.BlockSpec((SEQ, KC), lambda ni, ki: (0, ki)),
            pl.BlockSpec((NC, KC), lambda ni, ki: (ni, ki)),
        ],
        out_specs=pl.BlockSpec((SEQ, NC), lambda ni, ki: (0, ni)),
        out_shape=jax.ShapeDtypeStruct((SEQ, HIDDEN), jnp.float32),
        scratch_shapes=[pltpu.VMEM((SEQ, NC), jnp.float32)],
        compiler_params=pltpu.CompilerParams(
            dimension_semantics=("arbitrary", "arbitrary")),
        interpret=interpret,
    )(a, Wo)


def kernel(hidden_states, position_ids, Wq, Wk, Wv, Wo, interpret=False):
    xb, cos, ssin = _tables_call(hidden_states[0], interpret=interpret)
    q = _qproj_call(xb, Wq, cos, ssin, interpret=interpret)
    k, v = _kvproj_call(xb, Wk, Wv, cos, ssin, interpret=interpret)
    had = _hadamard_call(k, interpret=interpret)
    attn = _attn_call(q, k, v, interpret=interpret)
    out = _oproj_call(attn, Wo, interpret=interpret)
    return out[None], had.reshape(SEQ, N_KV, HD)
